# Initial kernel scaffold; baseline (speedup 1.0000x reference)
import jax
import jax.numpy as jnp
from jax.experimental import pallas as pl


def kernel(edge_index, X, initial_weight, W_ih, W_hh, b_ih, b_hh):
    raise NotImplementedError("write your pallas kernel here")


# speedup vs baseline: 21.4313x; 21.4313x over previous
import functools

import jax
import jax.numpy as jnp
from jax import lax
from jax.experimental import pallas as pl
from jax.experimental.pallas import tpu as pltpu
from jax.experimental.pallas import tpu_sc as plsc

N = 10000
E = 320000
D = 128
HALF = N // 2

NC = 2
NS = 16
NW = NC * NS
CHW = 80

EPW = E // NW
CH1 = EPW // CHW
EPT = E // NS
CH2 = EPT // CHW

TRASH = HALF
ACC_ROWS = HALF + 8
RPT = 320
RPT_LAST = HALF - RPT * (NS - 1)

DEG_RPT = 640
DEG_N = NS * DEG_RPT

_mesh = plsc.VectorSubcoreMesh(core_axis_name="c", subcore_axis_name="s")


@functools.partial(
    pl.kernel,
    out_type=jax.ShapeDtypeStruct((NC * DEG_N,), jnp.float32),
    mesh=_mesh,
    scratch_types=[
        pltpu.VMEM((CH1, CHW), jnp.int32),
        pltpu.VMEM((CHW,), jnp.float32),
        pltpu.VMEM((DEG_RPT,), jnp.float32),
        pltpu.VMEM_SHARED((DEG_N,), jnp.float32),
    ],
)
def _deg_kernel(col_hbm, deg_out, col_v, ones_v, zero_v, deg_acc):
    c = lax.axis_index("c")
    s = lax.axis_index("s")
    w = s * NC + c

    pltpu.sync_copy(col_hbm.at[w], col_v)
    for i in range(CHW // 16):
        ones_v[pl.ds(i * 16, 16)] = jnp.ones((16,), jnp.float32)
    for i in range(DEG_RPT // 16):
        zero_v[pl.ds(i * 16, 16)] = jnp.zeros((16,), jnp.float32)
    pltpu.sync_copy(zero_v, deg_acc.at[pl.ds(s * DEG_RPT, DEG_RPT)])
    plsc.subcore_barrier()

    def body(j, _):
        pltpu.sync_copy(ones_v, deg_acc.at[col_v.at[j]], add=True)
        return 0

    lax.fori_loop(0, CH1, body, 0)
    plsc.subcore_barrier()
    pltpu.sync_copy(deg_acc.at[pl.ds(s * DEG_RPT, DEG_RPT)],
                    deg_out.at[pl.ds(c * DEG_N + s * DEG_RPT, DEG_RPT)])


@functools.partial(
    pl.kernel,
    out_type=jax.ShapeDtypeStruct((NC, HALF, D), jnp.float32),
    mesh=_mesh,
    scratch_types=[
        pltpu.VMEM((CH2, CHW), jnp.int32),
        pltpu.VMEM((CH2, CHW), jnp.int32),
        pltpu.VMEM((CHW, D), jnp.float32),
        pltpu.VMEM((CHW, D), jnp.float32),
        pltpu.VMEM((8, D), jnp.float32),
        pltpu.VMEM_SHARED((ACC_ROWS, D), jnp.float32),
        pltpu.SemaphoreType.DMA,
        pltpu.SemaphoreType.DMA,
    ],
)
def _scatter_kernel(row_hbm, col_hbm, y_hbm, acc_out,
                    row_v, col_v, buf0, buf1, zbuf, acc, sem0, sem1):
    c = lax.axis_index("c")
    s = lax.axis_index("s")
    lo = c * HALF
    base = s * RPT

    pltpu.sync_copy(row_hbm.at[s], row_v)
    pltpu.sync_copy(col_hbm.at[s], col_v)

    @pl.when(s < NS - 1)
    def _():
        pltpu.sync_copy(y_hbm.at[pl.ds(lo + base, RPT)],
                        acc.at[pl.ds(base, RPT)])

    @pl.when(s == NS - 1)
    def _():
        pltpu.sync_copy(y_hbm.at[pl.ds(lo + base, RPT_LAST)],
                        acc.at[pl.ds(base, RPT_LAST)])
        for i in range(8):
            for j in range(D // 16):
                zbuf[i, pl.ds(j * 16, 16)] = jnp.zeros((16,), jnp.float32)
        pltpu.sync_copy(zbuf, acc.at[pl.ds(HALF, 8)])

    def localize(i, _):
        for j in range(CHW // 16):
            col16 = col_v[i, pl.ds(j * 16, 16)]
            local = col16 - lo
            owned = (local >= 0) & (local < HALF)
            col_v[i, pl.ds(j * 16, 16)] = jnp.where(owned, local, TRASH)
        return 0

    lax.fori_loop(0, CH2, localize, 0)
    plsc.subcore_barrier()

    pltpu.make_async_copy(y_hbm.at[row_v.at[0]], buf0, sem0).start()

    def body(i, _):
        j = 2 * i
        pltpu.make_async_copy(y_hbm.at[row_v.at[j]], buf0, sem0).wait()
        pltpu.make_async_copy(y_hbm.at[row_v.at[j + 1]], buf1, sem1).start()
        pltpu.sync_copy(buf0, acc.at[col_v.at[j]], add=True)
        pltpu.make_async_copy(y_hbm.at[row_v.at[j + 1]], buf1, sem1).wait()
        pltpu.make_async_copy(y_hbm.at[row_v.at[j + 2]], buf0, sem0).start()
        pltpu.sync_copy(buf1, acc.at[col_v.at[j + 1]], add=True)
        return 0

    lax.fori_loop(0, CH2 // 2 - 1, body, 0)
    pltpu.make_async_copy(y_hbm.at[row_v.at[CH2 - 2]], buf0, sem0).wait()
    pltpu.make_async_copy(y_hbm.at[row_v.at[CH2 - 1]], buf1, sem1).start()
    pltpu.sync_copy(buf0, acc.at[col_v.at[CH2 - 2]], add=True)
    pltpu.make_async_copy(y_hbm.at[row_v.at[CH2 - 1]], buf1, sem1).wait()
    pltpu.sync_copy(buf1, acc.at[col_v.at[CH2 - 1]], add=True)

    plsc.subcore_barrier()

    @pl.when(s < NS - 1)
    def _():
        pltpu.sync_copy(acc.at[pl.ds(base, RPT)],
                        acc_out.at[c].at[pl.ds(base, RPT)])

    @pl.when(s == NS - 1)
    def _():
        pltpu.sync_copy(acc.at[pl.ds(base, RPT_LAST)],
                        acc_out.at[c].at[pl.ds(base, RPT_LAST)])


def _gru_body(x0_ref, wih_ref, whh_ref, bih_ref, bhh_ref, w_ref):
    x0 = x0_ref[...]
    dn = (((1,), (1,)), ((), ()))
    gi = lax.dot_general(x0, wih_ref[...], dn,
                         preferred_element_type=jnp.float32) + bih_ref[...]
    gh = lax.dot_general(x0, whh_ref[...], dn,
                         preferred_element_type=jnp.float32) + bhh_ref[...]
    r = jax.nn.sigmoid(gi[:, 0:D] + gh[:, 0:D])
    z = jax.nn.sigmoid(gi[:, D:2 * D] + gh[:, D:2 * D])
    n = jnp.tanh(gi[:, 2 * D:3 * D] + r * gh[:, 2 * D:3 * D])
    w_ref[...] = (1.0 - z) * n + z * x0


_gru = pl.pallas_call(
    _gru_body,
    out_shape=jax.ShapeDtypeStruct((D, D), jnp.float32),
)


_YBLK = 1000


def _y_body(x_ref, w_ref, degt_ref, y_ref):
    dn = (((1,), (0,)), ((), ()))
    xw = lax.dot_general(x_ref[...], w_ref[...], dn,
                         preferred_element_type=jnp.float32)
    dp = degt_ref[...]
    dinv = lax.rsqrt(dp[:, 0:1] + dp[:, 1:2] + 1.0)
    y_ref[...] = dinv * xw


_y_call = pl.pallas_call(
    _y_body,
    grid=(N // _YBLK,),
    in_specs=[
        pl.BlockSpec((_YBLK, D), lambda i: (i, 0)),
        pl.BlockSpec((D, D), lambda i: (0, 0)),
        pl.BlockSpec((_YBLK, 2), lambda i: (i, 0)),
    ],
    out_specs=pl.BlockSpec((_YBLK, D), lambda i: (i, 0)),
    out_shape=jax.ShapeDtypeStruct((N, D), jnp.float32),
)


_CBLK = 1000


def _comb_body(acc_ref, degt_ref, out_ref):
    a = acc_ref[0]
    dp = degt_ref[...]
    dinv = lax.rsqrt(dp[:, 0:1] + dp[:, 1:2] + 1.0)
    out_ref[...] = dinv * a


_comb_call = pl.pallas_call(
    _comb_body,
    grid=(N // _CBLK,),
    in_specs=[
        pl.BlockSpec((1, _CBLK, D),
                     lambda i: (i // (HALF // _CBLK), i % (HALF // _CBLK), 0)),
        pl.BlockSpec((_CBLK, 2), lambda i: (i, 0)),
    ],
    out_specs=pl.BlockSpec((_CBLK, D), lambda i: (i, 0)),
    out_shape=jax.ShapeDtypeStruct((N, D), jnp.float32),
)


def kernel(edge_index, X, initial_weight, W_ih, W_hh, b_ih, b_hh):
    row_w = edge_index[0].astype(jnp.int32).reshape(NW, CH1, CHW)
    col_w = edge_index[1].astype(jnp.int32).reshape(NW, CH1, CHW)
    row_t = edge_index[0].astype(jnp.int32).reshape(NS, CH2, CHW)
    col_t = edge_index[1].astype(jnp.int32).reshape(NS, CH2, CHW)

    W = _gru(initial_weight[0], W_ih, W_hh,
             b_ih.reshape(1, 3 * D), b_hh.reshape(1, 3 * D))

    deg_parts = _deg_kernel(col_w)
    degt = deg_parts.reshape(NC, DEG_N)[:, :N].T

    y = _y_call(X, W, degt)
    acc = _scatter_kernel(row_t, col_t, y)
    out = _comb_call(acc, degt)
    return out
